# Initial kernel scaffold; baseline (speedup 1.0000x reference)
import jax
import jax.numpy as jnp
from jax.experimental import pallas as pl


def kernel(x, edge_index, W, b):
    raise NotImplementedError("write your pallas kernel here")


# speedup vs baseline: 2.4117x; 2.4117x over previous
import functools

import jax
import jax.numpy as jnp
from jax import lax
from jax.experimental import pallas as pl
from jax.experimental.pallas import tpu as pltpu
from jax.experimental.pallas import tpu_sc as plsc

NC, NS, LANES = 2, 16, 16
NT = NC * NS
F = 128
NPAD = 10240
RPT = NPAD // NT
ECHUNK = 8000
Q = 128
NEG = -3.0e38


def _matmul_body(x_ref, w1_ref, w2_ref, b_ref, a_ref, bm_ref):
    xb = x_ref[...]
    w2 = w2_ref[...]
    a_ref[...] = (
        jnp.dot(xb, w1_ref[...] - w2, preferred_element_type=jnp.float32)
        + b_ref[...]
    )
    bm_ref[...] = jnp.dot(xb, w2, preferred_element_type=jnp.float32)


def _tc_matmuls(xf, w1, w2, b2):
    blk = 1024
    return pl.pallas_call(
        _matmul_body,
        grid=(NPAD // blk,),
        in_specs=[
            pl.BlockSpec((blk, F), lambda i: (i, 0)),
            pl.BlockSpec((F, F), lambda i: (0, 0)),
            pl.BlockSpec((F, F), lambda i: (0, 0)),
            pl.BlockSpec((1, F), lambda i: (0, 0)),
        ],
        out_specs=[
            pl.BlockSpec((blk, F), lambda i: (i, 0)),
            pl.BlockSpec((blk, F), lambda i: (i, 0)),
        ],
        out_shape=[jax.ShapeDtypeStruct((NPAD, F), jnp.float32)] * 2,
    )(xf, w1, w2, b2)


def _segmax_body(a_hbm, bm_hbm, dst_hbm, src_hbm, out_hbm,
                 acc, dbuf0, dbuf1, sbuf0, sbuf1, qsrc, qdst, rows, afin,
                 sem_d0, sem_d1, sem_s0, sem_s1, sem_g):
    wid = lax.axis_index("s") * NC + lax.axis_index("c")
    lo = wid * RPT
    e_total = dst_hbm.shape[0]
    nch = e_total // ECHUNK
    iota = lax.iota(jnp.int32, LANES)
    dbufs = (dbuf0, dbuf1)
    sbufs = (sbuf0, sbuf1)
    sem_d = (sem_d0, sem_d1)
    sem_s = (sem_s0, sem_s1)

    neg16 = jnp.full((LANES,), NEG, jnp.float32)

    @pl.loop(0, (RPT + 1) * F // LANES)
    def _(i):
        acc[pl.ds(i * LANES, LANES)] = neg16

    def flush():
        pltpu.async_copy(bm_hbm.at[qsrc.at[pl.ds(0, Q)]], rows, sem_g).wait()

        @pl.loop(0, Q)
        def _(j):
            base = qdst[pl.ds(j, LANES)][0] * F
            for c in range(F // LANES):
                off = base + c * LANES
                acc[pl.ds(off, LANES)] = jnp.maximum(
                    acc[pl.ds(off, LANES)], rows[j, pl.ds(c * LANES, LANES)]
                )

    def start_chunk(ci, b):
        pltpu.async_copy(dst_hbm.at[pl.ds(ci * ECHUNK, ECHUNK)], dbufs[b],
                         sem_d[b])
        pltpu.async_copy(src_hbm.at[pl.ds(ci * ECHUNK, ECHUNK)], sbufs[b],
                         sem_s[b])

    def wait_chunk(ci, b):
        pltpu.make_async_copy(dst_hbm.at[pl.ds(ci * ECHUNK, ECHUNK)],
                              dbufs[b], sem_d[b]).wait()
        pltpu.make_async_copy(src_hbm.at[pl.ds(ci * ECHUNK, ECHUNK)],
                              sbufs[b], sem_s[b]).wait()

    start_chunk(0, 0)

    def do_chunk(ci, b, qn):
        wait_chunk(ci, b)

        @pl.when(ci + 1 < nch)
        def _():
            start_chunk(ci + 1, 1 - b)

        @pl.loop(0, ECHUNK // LANES, init_carry=qn)
        def qn(g, qn):
            dv = dbufs[b][pl.ds(g * LANES, LANES)]
            sv = sbufs[b][pl.ds(g * LANES, LANES)]
            dl = dv - lo
            m = (dl >= 0) & (dl < RPT)
            cnt = plsc.all_reduce_population_count(m)[0]
            plsc.store_compressed(qsrc.at[pl.ds(qn, LANES)], sv, mask=m)
            plsc.store_compressed(qdst.at[pl.ds(qn, LANES)], dl, mask=m)
            qn = qn + cnt

            @pl.when(qn >= Q)
            def _():
                flush()
                ts = qsrc[pl.ds(Q, LANES)]
                td = qdst[pl.ds(Q, LANES)]
                qsrc[pl.ds(0, LANES)] = ts
                qdst[pl.ds(0, LANES)] = td

            return jnp.where(qn >= Q, qn - Q, qn)

        return qn

    @pl.loop(0, nch // 2, init_carry=0)
    def qn(i, qn):
        qn = do_chunk(2 * i, 0, qn)
        qn = do_chunk(2 * i + 1, 1, qn)
        return qn

    for gi in range(Q // LANES):
        pos = iota + gi * LANES
        m = pos >= qn
        sv = qsrc[pl.ds(gi * LANES, LANES)]
        dv = qdst[pl.ds(gi * LANES, LANES)]
        qsrc[pl.ds(gi * LANES, LANES)] = jnp.where(m, 0, sv)
        qdst[pl.ds(gi * LANES, LANES)] = jnp.where(m, RPT, dv)
    flush()

    @pl.loop(0, RPT // LANES)
    def _(r):
        pltpu.sync_copy(a_hbm.at[pl.ds(lo + r * LANES, LANES)], afin)
        for rr in range(LANES):
            for c in range(F // LANES):
                off = (r * LANES + rr) * F + c * LANES
                v = afin[rr, pl.ds(c * LANES, LANES)] + acc[pl.ds(off, LANES)]
                afin[rr, pl.ds(c * LANES, LANES)] = jnp.maximum(v, 0.0)
        pltpu.sync_copy(afin, out_hbm.at[pl.ds(lo + r * LANES, LANES)])


def _sc_segmax(a, bm, dst, src):
    mesh = plsc.VectorSubcoreMesh(
        core_axis_name="c", subcore_axis_name="s",
        num_cores=NC, num_subcores=NS,
    )
    call = functools.partial(
        pl.kernel,
        out_type=jax.ShapeDtypeStruct((NPAD, F), jnp.float32),
        mesh=mesh,
        compiler_params=pltpu.CompilerParams(needs_layout_passes=False),
        scratch_types=[
            pltpu.VMEM(((RPT + 1) * F,), jnp.float32),
            pltpu.VMEM((ECHUNK,), jnp.int32),
            pltpu.VMEM((ECHUNK,), jnp.int32),
            pltpu.VMEM((ECHUNK,), jnp.int32),
            pltpu.VMEM((ECHUNK,), jnp.int32),
            pltpu.VMEM((Q + LANES,), jnp.int32),
            pltpu.VMEM((Q + LANES,), jnp.int32),
            pltpu.VMEM((Q, F), jnp.float32),
            pltpu.VMEM((LANES, F), jnp.float32),
            pltpu.SemaphoreType.DMA,
            pltpu.SemaphoreType.DMA,
            pltpu.SemaphoreType.DMA,
            pltpu.SemaphoreType.DMA,
            pltpu.SemaphoreType.DMA,
        ],
    )(_segmax_body)
    return call(a, bm, dst, src)


@jax.jit
def kernel(x, edge_index, W, b):
    bx, cx, nx, _ = x.shape
    k = edge_index.shape[-1]
    xf = jnp.transpose(x[..., 0], (0, 2, 1)).reshape(bx * nx, cx)
    xf = jnp.pad(xf, ((0, NPAD - bx * nx), (0, 0)))
    ei = edge_index.reshape(2, bx, nx * k)
    offsets = (jnp.arange(bx, dtype=edge_index.dtype) * nx)[None, :, None]
    ei = (ei + offsets).reshape(2, -1).astype(jnp.int32)
    src, dst = ei[0], ei[1]

    a, bm = _tc_matmuls(xf, W[:cx], W[cx:], b.reshape(1, F))
    out_full = _sc_segmax(a, bm, dst, src)
    out = out_full[: bx * nx]
    return jnp.transpose(out.reshape(bx, nx, F), (0, 2, 1))[..., None]


# speedup vs baseline: 2.7742x; 1.1503x over previous
import functools

import jax
import jax.numpy as jnp
from jax import lax
from jax.experimental import pallas as pl
from jax.experimental.pallas import tpu as pltpu
from jax.experimental.pallas import tpu_sc as plsc

NC, NS, LANES = 2, 16, 16
NT = NC * NS
F = 128
NPAD = 10240
RPT = NPAD // NT
ECHUNK = 8000
Q = 128
NEG = -3.0e38


def _matmul_body(x_ref, w1_ref, w2_ref, b_ref, a_ref, bm_ref):
    xb = x_ref[...]
    w2 = w2_ref[...]
    a_ref[...] = (
        jnp.dot(xb, w1_ref[...] - w2, preferred_element_type=jnp.float32)
        + b_ref[...]
    )
    bm_ref[...] = jnp.dot(xb, w2, preferred_element_type=jnp.float32)


def _tc_matmuls(xf, w1, w2, b2):
    blk = 1024
    return pl.pallas_call(
        _matmul_body,
        grid=(NPAD // blk,),
        in_specs=[
            pl.BlockSpec((blk, F), lambda i: (i, 0)),
            pl.BlockSpec((F, F), lambda i: (0, 0)),
            pl.BlockSpec((F, F), lambda i: (0, 0)),
            pl.BlockSpec((1, F), lambda i: (0, 0)),
        ],
        out_specs=[
            pl.BlockSpec((blk, F), lambda i: (i, 0)),
            pl.BlockSpec((blk, F), lambda i: (i, 0)),
        ],
        out_shape=[jax.ShapeDtypeStruct((NPAD, F), jnp.float32)] * 2,
    )(xf, w1, w2, b2)


def _segmax_body(a_hbm, bm_hbm, dst_hbm, src_hbm, out_hbm,
                 acc, dbuf0, dbuf1, sbuf0, sbuf1, qsrc, qdst,
                 qsrcs0, qsrcs1, qdsts0, qdsts1, rows0, rows1, afin,
                 sem_d0, sem_d1, sem_s0, sem_s1, sem_g0, sem_g1):
    wid = lax.axis_index("s") * NC + lax.axis_index("c")
    lo = wid * RPT
    e_total = dst_hbm.shape[0]
    nch = e_total // ECHUNK
    iota = lax.iota(jnp.int32, LANES)
    dbufs = (dbuf0, dbuf1)
    sbufs = (sbuf0, sbuf1)
    sem_d = (sem_d0, sem_d1)
    sem_s = (sem_s0, sem_s1)

    neg16 = jnp.full((LANES,), NEG, jnp.float32)
    qsrcs = (qsrcs0, qsrcs1)
    qdsts = (qdsts0, qdsts1)
    rowss = (rows0, rows1)
    sem_g = (sem_g0, sem_g1)

    @pl.loop(0, (RPT + 1) * F // LANES, unroll=4)
    def _(i):
        acc[pl.ds(i * LANES, LANES)] = neg16

    def stage_and_start(s):
        for gi in range(Q // LANES):
            qsrcs[s][pl.ds(gi * LANES, LANES)] = qsrc[pl.ds(gi * LANES, LANES)]
            qdsts[s][pl.ds(gi * LANES, LANES)] = qdst[pl.ds(gi * LANES, LANES)]
        pltpu.async_copy(bm_hbm.at[qsrcs[s].at[pl.ds(0, Q)]], rowss[s],
                         sem_g[s])

    def drain(s):
        pltpu.make_async_copy(bm_hbm.at[qsrcs[s].at[pl.ds(0, Q)]], rowss[s],
                              sem_g[s]).wait()
        qd = qdsts[s]
        rw = rowss[s]

        @pl.loop(0, Q, unroll=2)
        def _(j):
            base = qd[pl.ds(j, LANES)][0] * F
            for c in range(F // LANES):
                off = base + c * LANES
                acc[pl.ds(off, LANES)] = jnp.maximum(
                    acc[pl.ds(off, LANES)], rw[j, pl.ds(c * LANES, LANES)]
                )

    def flush_event(fc):
        par = fc & 1

        @pl.when(par == 0)
        def _():
            stage_and_start(0)

            @pl.when(fc > 0)
            def _():
                drain(1)

        @pl.when(par == 1)
        def _():
            stage_and_start(1)
            drain(0)

    def start_chunk(ci, b):
        pltpu.async_copy(dst_hbm.at[pl.ds(ci * ECHUNK, ECHUNK)], dbufs[b],
                         sem_d[b])
        pltpu.async_copy(src_hbm.at[pl.ds(ci * ECHUNK, ECHUNK)], sbufs[b],
                         sem_s[b])

    def wait_chunk(ci, b):
        pltpu.make_async_copy(dst_hbm.at[pl.ds(ci * ECHUNK, ECHUNK)],
                              dbufs[b], sem_d[b]).wait()
        pltpu.make_async_copy(src_hbm.at[pl.ds(ci * ECHUNK, ECHUNK)],
                              sbufs[b], sem_s[b]).wait()

    start_chunk(0, 0)

    def do_chunk(ci, b, carry):
        wait_chunk(ci, b)

        @pl.when(ci + 1 < nch)
        def _():
            start_chunk(ci + 1, 1 - b)

        @pl.loop(0, ECHUNK // LANES, init_carry=carry, unroll=2)
        def carry(g, carry):
            qn, fc = carry
            dv = dbufs[b][pl.ds(g * LANES, LANES)]
            sv = sbufs[b][pl.ds(g * LANES, LANES)]
            dl = dv - lo
            m = (dl >= 0) & (dl < RPT)
            cnt = plsc.all_reduce_population_count(m)[0]
            plsc.store_compressed(qsrc.at[pl.ds(qn, LANES)], sv, mask=m)
            plsc.store_compressed(qdst.at[pl.ds(qn, LANES)], dl, mask=m)
            qn = qn + cnt

            @pl.when(qn >= Q)
            def _():
                flush_event(fc)
                ts = qsrc[pl.ds(Q, LANES)]
                td = qdst[pl.ds(Q, LANES)]
                qsrc[pl.ds(0, LANES)] = ts
                qdst[pl.ds(0, LANES)] = td

            full = qn >= Q
            return (jnp.where(full, qn - Q, qn), jnp.where(full, fc + 1, fc))

        return carry

    @pl.loop(0, nch // 2, init_carry=(0, 0))
    def carry(i, carry):
        carry = do_chunk(2 * i, 0, carry)
        carry = do_chunk(2 * i + 1, 1, carry)
        return carry

    qn, fc = carry
    for gi in range(Q // LANES):
        pos = iota + gi * LANES
        m = pos >= qn
        sv = qsrc[pl.ds(gi * LANES, LANES)]
        dv = qdst[pl.ds(gi * LANES, LANES)]
        qsrc[pl.ds(gi * LANES, LANES)] = jnp.where(m, 0, sv)
        qdst[pl.ds(gi * LANES, LANES)] = jnp.where(m, RPT, dv)
    flush_event(fc)
    par = fc & 1

    @pl.when(par == 0)
    def _():
        drain(0)

    @pl.when(par == 1)
    def _():
        drain(1)

    @pl.loop(0, RPT // LANES)
    def _(r):
        pltpu.sync_copy(a_hbm.at[pl.ds(lo + r * LANES, LANES)], afin)
        for rr in range(LANES):
            for c in range(F // LANES):
                off = (r * LANES + rr) * F + c * LANES
                v = afin[rr, pl.ds(c * LANES, LANES)] + acc[pl.ds(off, LANES)]
                afin[rr, pl.ds(c * LANES, LANES)] = jnp.maximum(v, 0.0)
        pltpu.sync_copy(afin, out_hbm.at[pl.ds(lo + r * LANES, LANES)])


def _sc_segmax(a, bm, dst, src):
    mesh = plsc.VectorSubcoreMesh(
        core_axis_name="c", subcore_axis_name="s",
        num_cores=NC, num_subcores=NS,
    )
    call = functools.partial(
        pl.kernel,
        out_type=jax.ShapeDtypeStruct((NPAD, F), jnp.float32),
        mesh=mesh,
        compiler_params=pltpu.CompilerParams(needs_layout_passes=False),
        scratch_types=[
            pltpu.VMEM(((RPT + 1) * F,), jnp.float32),
            pltpu.VMEM((ECHUNK,), jnp.int32),
            pltpu.VMEM((ECHUNK,), jnp.int32),
            pltpu.VMEM((ECHUNK,), jnp.int32),
            pltpu.VMEM((ECHUNK,), jnp.int32),
            pltpu.VMEM((Q + LANES,), jnp.int32),
            pltpu.VMEM((Q + LANES,), jnp.int32),
            pltpu.VMEM((Q,), jnp.int32),
            pltpu.VMEM((Q,), jnp.int32),
            pltpu.VMEM((Q + LANES,), jnp.int32),
            pltpu.VMEM((Q + LANES,), jnp.int32),
            pltpu.VMEM((Q, F), jnp.float32),
            pltpu.VMEM((Q, F), jnp.float32),
            pltpu.VMEM((LANES, F), jnp.float32),
            pltpu.SemaphoreType.DMA,
            pltpu.SemaphoreType.DMA,
            pltpu.SemaphoreType.DMA,
            pltpu.SemaphoreType.DMA,
            pltpu.SemaphoreType.DMA,
            pltpu.SemaphoreType.DMA,
        ],
    )(_segmax_body)
    return call(a, bm, dst, src)


@jax.jit
def kernel(x, edge_index, W, b):
    bx, cx, nx, _ = x.shape
    k = edge_index.shape[-1]
    xf = jnp.transpose(x[..., 0], (0, 2, 1)).reshape(bx * nx, cx)
    xf = jnp.pad(xf, ((0, NPAD - bx * nx), (0, 0)))
    ei = edge_index.reshape(2, bx, nx * k)
    offsets = (jnp.arange(bx, dtype=edge_index.dtype) * nx)[None, :, None]
    ei = (ei + offsets).reshape(2, -1).astype(jnp.int32)
    src, dst = ei[0], ei[1]

    a, bm = _tc_matmuls(xf, W[:cx], W[cx:], b.reshape(1, F))
    out_full = _sc_segmax(a, bm, dst, src)
    out = out_full[: bx * nx]
    return jnp.transpose(out.reshape(bx, nx, F), (0, 2, 1))[..., None]
